# Initial kernel scaffold; baseline (speedup 1.0000x reference)
import jax
import jax.numpy as jnp
from jax.experimental import pallas as pl


def kernel(inputs, targets, mask):
    raise NotImplementedError("write your pallas kernel here")


# speedup vs baseline: 36.3198x; 36.3198x over previous
import functools

import jax
import jax.numpy as jnp
from jax import lax
from jax.experimental import pallas as pl
from jax.experimental.pallas import tpu as pltpu
from jax.experimental.pallas import tpu_sc as plsc

N_TOTAL = 64 * 512 * 512
ROWS, COLS = 16384, 1024
BLK_ROWS = 512
GRID = ROWS // BLK_ROWS

NBINS = 65536
HIST_R, HIST_C = 512, 128

NW = 32
PER_W = N_TOTAL // NW
SC_CHUNK = 32768
SC_CHUNKS = PER_W // SC_CHUNK


def _loss_body(x_ref, t_ref, m_ref, loss_ref, n_ref):
    @pl.when(pl.program_id(0) == 0)
    def _():
        n_ref[0, 0] = 0

    d = x_ref[...] - t_ref[...]
    ad = jnp.abs(d)
    sl1 = jnp.where(ad < 1.0, 0.5 * d * d, ad - 0.5)
    m = m_ref[...]
    loss_ref[...] = lax.bitcast_convert_type(jnp.where(m, sl1, 0.0), jnp.int32)
    n_ref[0, 0] += jnp.sum(m.astype(jnp.int32))


def _loss_pass(x, t, m):
    return pl.pallas_call(
        _loss_body,
        grid=(GRID,),
        in_specs=[
            pl.BlockSpec((BLK_ROWS, COLS), lambda i: (i, 0)),
            pl.BlockSpec((BLK_ROWS, COLS), lambda i: (i, 0)),
            pl.BlockSpec((BLK_ROWS, COLS), lambda i: (i, 0)),
        ],
        out_specs=[
            pl.BlockSpec((BLK_ROWS, COLS), lambda i: (i, 0)),
            pl.BlockSpec((1, 1), lambda i: (0, 0), memory_space=pltpu.SMEM),
        ],
        out_shape=[
            jax.ShapeDtypeStruct((ROWS, COLS), jnp.int32),
            jax.ShapeDtypeStruct((1, 1), jnp.int32),
        ],
    )(x, t, m)


def _hist_sc_body(loss_hbm, out_hbm, hist, buf, sem):
    wid = lax.axis_index("s") * 2 + lax.axis_index("c")
    base = wid * PER_W

    zeros16 = jnp.zeros((16,), jnp.int32)

    def zbody(i, carry):
        hist[pl.ds(i * 16, 16)] = zeros16
        return carry

    lax.fori_loop(0, NBINS // 16, zbody, 0)

    ones16 = jnp.ones((16,), jnp.int32)
    shift16 = jnp.full((16,), 16, jnp.int32)

    def cbody(c, carry):
        pltpu.sync_copy(loss_hbm.at[pl.ds(base + c * SC_CHUNK, SC_CHUNK)], buf)

        def vbody(j, inner):
            bits = buf[pl.ds(j * 16, 16)]
            bins = lax.shift_right_logical(bits, shift16)
            plsc.addupdate_scatter(hist, [bins], ones16)
            return inner

        lax.fori_loop(0, SC_CHUNK // 16, vbody, 0)
        return carry

    lax.fori_loop(0, SC_CHUNKS, cbody, 0)

    pltpu.sync_copy(hist, out_hbm.at[wid])


def _hist_pass(loss_flat):
    mesh = plsc.VectorSubcoreMesh(core_axis_name="c", subcore_axis_name="s")
    kern = pl.kernel(
        _hist_sc_body,
        out_type=jax.ShapeDtypeStruct((NW, NBINS), jnp.int32),
        mesh=mesh,
        compiler_params=pltpu.CompilerParams(needs_layout_passes=False),
        scratch_types=[
            pltpu.VMEM((NBINS,), jnp.int32),
            pltpu.VMEM((SC_CHUNK,), jnp.int32),
            pltpu.SemaphoreType.DMA,
        ],
    )
    return kern(loss_flat)


def _final_body(hist_ref, n_ref, loss_ref, out_ref, s_ref, f_ref):
    i = pl.program_id(0)

    @pl.when(i == 0)
    def _():
        cnt = jnp.sum(hist_ref[...], axis=0).astype(jnp.int32)
        n = n_ref[0, 0]
        k = (6 * n) // 10

        binid = (lax.broadcasted_iota(jnp.int32, (HIST_R, HIST_C), 0) * HIST_C
                 + lax.broadcasted_iota(jnp.int32, (HIST_R, HIST_C), 1))

        def bs_body(_, lo_hi):
            lo, hi = lo_hi
            mid = (lo + hi) // 2
            s_mid = jnp.sum(jnp.where(binid >= mid, cnt, 0))
            return jnp.where(s_mid >= k, mid, lo), jnp.where(s_mid >= k, hi, mid)

        sel, _ = lax.fori_loop(0, 16, bs_body, (jnp.int32(0), jnp.int32(NBINS)))
        c_hi = jnp.sum(jnp.where(binid > sel, cnt, 0))
        m = jnp.sum(jnp.where(binid == sel, cnt, 0))

        s_ref[0] = sel
        s_ref[1] = c_hi
        s_ref[2] = m
        s_ref[3] = k
        f_ref[0] = 0.0
        f_ref[1] = 0.0

    sel = s_ref[0]
    bits = loss_ref[...]
    loss = lax.bitcast_convert_type(bits, jnp.float32)
    bins = lax.shift_right_logical(bits, 16)
    f_ref[0] += jnp.sum(jnp.where(bins > sel, loss, 0.0))
    f_ref[1] += jnp.sum(jnp.where(bins == sel, loss, 0.0))

    @pl.when(i == pl.num_programs(0) - 1)
    def _():
        c_hi, m, k = s_ref[1], s_ref[2], s_ref[3]
        r = jnp.clip(k - c_hi, 0, m)
        mean_in = jnp.where(m > 0, f_ref[1] / m.astype(jnp.float32), 0.0)
        out_ref[0, 0] = ((f_ref[0] + r.astype(jnp.float32) * mean_in)
                         / k.astype(jnp.float32))


def _final_pass(hist, n, loss):
    hist3 = hist.reshape(NW, HIST_R, HIST_C)
    return pl.pallas_call(
        _final_body,
        grid=(GRID,),
        in_specs=[
            pl.BlockSpec((NW, HIST_R, HIST_C), lambda i: (0, 0, 0)),
            pl.BlockSpec((1, 1), lambda i: (0, 0), memory_space=pltpu.SMEM),
            pl.BlockSpec((BLK_ROWS, COLS), lambda i: (i, 0)),
        ],
        out_specs=pl.BlockSpec((1, 1), lambda i: (0, 0), memory_space=pltpu.SMEM),
        out_shape=jax.ShapeDtypeStruct((1, 1), jnp.float32),
        scratch_shapes=[
            pltpu.SMEM((4,), jnp.int32),
            pltpu.SMEM((2,), jnp.float32),
        ],
    )(hist3, n, loss)


def kernel(inputs, targets, mask):
    x = inputs.reshape(ROWS, COLS)
    t = targets.reshape(ROWS, COLS)
    m = mask.reshape(ROWS, COLS)
    loss, n = _loss_pass(x, t, m)
    hist = _hist_pass(loss.reshape(N_TOTAL))
    out = _final_pass(hist, n, loss)
    return out.reshape(())


# speedup vs baseline: 52.8863x; 1.4561x over previous
import functools

import jax
import jax.numpy as jnp
from jax import lax
from jax.experimental import pallas as pl
from jax.experimental.pallas import tpu as pltpu
from jax.experimental.pallas import tpu_sc as plsc

N_TOTAL = 64 * 512 * 512
ROWS, COLS = 32768, 512
GRID = 32
BLK3 = (2, 512, 512)
BLK_ROWS = 1024

NBINS = 65536
HIST_R, HIST_C = 512, 128

NW = 32
ROWS_W = ROWS // NW
CHUNK_R = 32
CHUNKS = ROWS_W // CHUNK_R


def _loss_body(x_ref, t_ref, m_ref, loss_ref, n_ref):
    @pl.when(pl.program_id(0) == 0)
    def _():
        n_ref[0, 0] = 0

    d = x_ref[...] - t_ref[...]
    ad = jnp.abs(d)
    sl1 = jnp.where(ad < 1.0, 0.5 * d * d, ad - 0.5)
    m = m_ref[...]
    bits = lax.bitcast_convert_type(jnp.where(m, sl1, 0.0), jnp.int32)
    loss_ref[...] = bits.reshape(BLK_ROWS, COLS)
    n_ref[0, 0] += jnp.sum(m.astype(jnp.int32))


def _loss_pass(x, t, m):
    return pl.pallas_call(
        _loss_body,
        grid=(GRID,),
        in_specs=[
            pl.BlockSpec(BLK3, lambda i: (i, 0, 0)),
            pl.BlockSpec(BLK3, lambda i: (i, 0, 0)),
            pl.BlockSpec(BLK3, lambda i: (i, 0, 0)),
        ],
        out_specs=[
            pl.BlockSpec((BLK_ROWS, COLS), lambda i: (i, 0)),
            pl.BlockSpec((1, 1), lambda i: (0, 0), memory_space=pltpu.SMEM),
        ],
        out_shape=[
            jax.ShapeDtypeStruct((ROWS, COLS), jnp.int32),
            jax.ShapeDtypeStruct((1, 1), jnp.int32),
        ],
    )(x, t, m)


def _hist_sc_body(loss_hbm, out_hbm, hist, buf0, buf1, sem0, sem1):
    wid = lax.axis_index("s") * 2 + lax.axis_index("c")
    row0 = wid * ROWS_W

    zeros16 = jnp.zeros((16,), jnp.int32)

    def zbody(i, carry):
        hist[pl.ds(i * 16, 16)] = zeros16
        return carry

    lax.fori_loop(0, NBINS // 16, zbody, 0)

    ones16 = jnp.ones((16,), jnp.int32)
    shift16 = jnp.full((16,), 16, jnp.int32)

    def chunk_at(c):
        return loss_hbm.at[pl.ds(row0 + c * CHUNK_R, CHUNK_R)]

    def process(buf):
        def rbody(r, carry):
            for u in range(COLS // 16):
                bits = buf[r, pl.ds(u * 16, 16)]
                bins = lax.shift_right_logical(bits, shift16)
                plsc.addupdate_scatter(hist, [bins], ones16)
            return carry

        lax.fori_loop(0, CHUNK_R, rbody, 0)

    pltpu.async_copy(chunk_at(0), buf0, sem0)

    def cbody(i, carry):
        c = 2 * i
        pltpu.async_copy(chunk_at(c + 1), buf1, sem1)
        pltpu.make_async_copy(chunk_at(c), buf0, sem0).wait()
        process(buf0)

        @pl.when(c + 2 < CHUNKS)
        def _():
            pltpu.async_copy(chunk_at(c + 2), buf0, sem0)

        pltpu.make_async_copy(chunk_at(c + 1), buf1, sem1).wait()
        process(buf1)
        return carry

    lax.fori_loop(0, CHUNKS // 2, cbody, 0)

    pltpu.sync_copy(hist, out_hbm.at[wid])


def _hist_pass(loss2d):
    mesh = plsc.VectorSubcoreMesh(core_axis_name="c", subcore_axis_name="s")
    kern = pl.kernel(
        _hist_sc_body,
        out_type=jax.ShapeDtypeStruct((NW, NBINS), jnp.int32),
        mesh=mesh,
        compiler_params=pltpu.CompilerParams(
            needs_layout_passes=False, use_tc_tiling_on_sc=False),
        scratch_types=[
            pltpu.VMEM((NBINS,), jnp.int32),
            pltpu.VMEM((CHUNK_R, COLS), jnp.int32),
            pltpu.VMEM((CHUNK_R, COLS), jnp.int32),
            pltpu.SemaphoreType.DMA,
            pltpu.SemaphoreType.DMA,
        ],
    )
    return kern(loss2d)


def _final_body(hist_ref, n_ref, loss_ref, out_ref, s_ref, f_ref):
    i = pl.program_id(0)

    @pl.when(i == 0)
    def _():
        cnt = jnp.sum(hist_ref[...], axis=0).astype(jnp.int32)
        n = n_ref[0, 0]
        k = (6 * n) // 10

        binid = (lax.broadcasted_iota(jnp.int32, (HIST_R, HIST_C), 0) * HIST_C
                 + lax.broadcasted_iota(jnp.int32, (HIST_R, HIST_C), 1))

        def bs_body(_, lo_hi):
            lo, hi = lo_hi
            mid = (lo + hi) // 2
            s_mid = jnp.sum(jnp.where(binid >= mid, cnt, 0))
            return jnp.where(s_mid >= k, mid, lo), jnp.where(s_mid >= k, hi, mid)

        sel, _ = lax.fori_loop(0, 16, bs_body, (jnp.int32(0), jnp.int32(NBINS)))
        c_hi = jnp.sum(jnp.where(binid > sel, cnt, 0))
        m = jnp.sum(jnp.where(binid == sel, cnt, 0))

        s_ref[0] = sel
        s_ref[1] = c_hi
        s_ref[2] = m
        s_ref[3] = k
        f_ref[0] = 0.0
        f_ref[1] = 0.0

    sel = s_ref[0]
    bits = loss_ref[...]
    loss = lax.bitcast_convert_type(bits, jnp.float32)
    bins = lax.shift_right_logical(bits, 16)
    f_ref[0] += jnp.sum(jnp.where(bins > sel, loss, 0.0))
    f_ref[1] += jnp.sum(jnp.where(bins == sel, loss, 0.0))

    @pl.when(i == pl.num_programs(0) - 1)
    def _():
        c_hi, m, k = s_ref[1], s_ref[2], s_ref[3]
        r = jnp.clip(k - c_hi, 0, m)
        mean_in = jnp.where(m > 0, f_ref[1] / m.astype(jnp.float32), 0.0)
        out_ref[0, 0] = ((f_ref[0] + r.astype(jnp.float32) * mean_in)
                         / k.astype(jnp.float32))


def _final_pass(hist, n, loss2d):
    hist3 = hist.reshape(NW, HIST_R, HIST_C)
    return pl.pallas_call(
        _final_body,
        grid=(GRID,),
        in_specs=[
            pl.BlockSpec((NW, HIST_R, HIST_C), lambda i: (0, 0, 0)),
            pl.BlockSpec((1, 1), lambda i: (0, 0), memory_space=pltpu.SMEM),
            pl.BlockSpec((BLK_ROWS, COLS), lambda i: (i, 0)),
        ],
        out_specs=pl.BlockSpec((1, 1), lambda i: (0, 0), memory_space=pltpu.SMEM),
        out_shape=jax.ShapeDtypeStruct((1, 1), jnp.float32),
        scratch_shapes=[
            pltpu.SMEM((4,), jnp.int32),
            pltpu.SMEM((2,), jnp.float32),
        ],
    )(hist3, n, loss2d)


def kernel(inputs, targets, mask):
    loss, n = _loss_pass(inputs, targets, mask)
    hist = _hist_pass(loss)
    out = _final_pass(hist, n, loss)
    return out.reshape(())


# speedup vs baseline: 58.5688x; 1.1074x over previous
import functools

import jax
import jax.numpy as jnp
from jax import lax
from jax.experimental import pallas as pl
from jax.experimental.pallas import tpu as pltpu
from jax.experimental.pallas import tpu_sc as plsc

N_TOTAL = 64 * 512 * 512
ROWS, COLS = 32768, 512
GRID = 32
BLK3 = (2, 512, 512)
BLK_ROWS = 1024

NBINS = 65536
HIST_R, HIST_C = 512, 128

NW = 32
ROWS_W = ROWS // NW
CHUNK_R = 32
CHUNKS = ROWS_W // CHUNK_R


def _loss_body(x_ref, t_ref, m_ref, loss_ref, n_ref):
    @pl.when(pl.program_id(0) == 0)
    def _():
        n_ref[0, 0] = 0

    d = x_ref[...] - t_ref[...]
    ad = jnp.abs(d)
    sl1 = jnp.where(ad < 1.0, 0.5 * d * d, ad - 0.5)
    m = m_ref[...]
    bits = lax.bitcast_convert_type(jnp.where(m, sl1, 0.0), jnp.int32)
    loss_ref[...] = bits.reshape(BLK_ROWS, COLS)
    n_ref[0, 0] += jnp.sum(m.astype(jnp.int32))


def _loss_pass(x, t, m):
    return pl.pallas_call(
        _loss_body,
        grid=(GRID,),
        in_specs=[
            pl.BlockSpec(BLK3, lambda i: (i, 0, 0)),
            pl.BlockSpec(BLK3, lambda i: (i, 0, 0)),
            pl.BlockSpec(BLK3, lambda i: (i, 0, 0)),
        ],
        out_specs=[
            pl.BlockSpec((BLK_ROWS, COLS), lambda i: (i, 0)),
            pl.BlockSpec((1, 1), lambda i: (0, 0), memory_space=pltpu.SMEM),
        ],
        out_shape=[
            jax.ShapeDtypeStruct((ROWS, COLS), jnp.int32),
            jax.ShapeDtypeStruct((1, 1), jnp.int32),
        ],
    )(x, t, m)


def _hist_sc_body(loss_hbm, out_hbm, hist, buf0, buf1, sem0, sem1):
    wid = lax.axis_index("s") * 2 + lax.axis_index("c")
    row0 = wid * ROWS_W

    zeros16 = jnp.zeros((16,), jnp.int32)

    def zbody(i, carry):
        hist[pl.ds(i * 16, 16)] = zeros16
        return carry

    lax.fori_loop(0, NBINS // 16, zbody, 0)

    ones16 = jnp.ones((16,), jnp.int32)
    shift16 = jnp.full((16,), 16, jnp.int32)
    mask16 = jnp.full((16,), 0xFFF8, jnp.int32)
    lane3 = jnp.bitwise_and(lax.iota(jnp.int32, 16), 7)

    def chunk_at(c):
        return loss_hbm.at[pl.ds(row0 + c * CHUNK_R, CHUNK_R)]

    def process(buf):
        def rbody(r, carry):
            for u in range(COLS // 16):
                bits = buf[r, pl.ds(u * 16, 16)]
                addr = jnp.bitwise_or(
                    jnp.bitwise_and(lax.shift_right_logical(bits, shift16),
                                    mask16),
                    lane3)
                plsc.addupdate_scatter(hist, [addr], ones16)
            return carry

        lax.fori_loop(0, CHUNK_R, rbody, 0)

    pltpu.async_copy(chunk_at(0), buf0, sem0)

    def cbody(i, carry):
        c = 2 * i
        pltpu.async_copy(chunk_at(c + 1), buf1, sem1)
        pltpu.make_async_copy(chunk_at(c), buf0, sem0).wait()
        process(buf0)

        @pl.when(c + 2 < CHUNKS)
        def _():
            pltpu.async_copy(chunk_at(c + 2), buf0, sem0)

        pltpu.make_async_copy(chunk_at(c + 1), buf1, sem1).wait()
        process(buf1)
        return carry

    lax.fori_loop(0, CHUNKS // 2, cbody, 0)

    pltpu.sync_copy(hist, out_hbm.at[wid])


def _hist_pass(loss2d):
    mesh = plsc.VectorSubcoreMesh(core_axis_name="c", subcore_axis_name="s")
    kern = pl.kernel(
        _hist_sc_body,
        out_type=jax.ShapeDtypeStruct((NW, NBINS), jnp.int32),
        mesh=mesh,
        compiler_params=pltpu.CompilerParams(
            needs_layout_passes=False, use_tc_tiling_on_sc=False),
        scratch_types=[
            pltpu.VMEM((NBINS,), jnp.int32),
            pltpu.VMEM((CHUNK_R, COLS), jnp.int32),
            pltpu.VMEM((CHUNK_R, COLS), jnp.int32),
            pltpu.SemaphoreType.DMA,
            pltpu.SemaphoreType.DMA,
        ],
    )
    return kern(loss2d)


def _final_body(hist_ref, n_ref, loss_ref, out_ref, s_ref, f_ref):
    i = pl.program_id(0)

    @pl.when(i == 0)
    def _():
        cnt = jnp.sum(hist_ref[...], axis=0).astype(jnp.int32)
        n = n_ref[0, 0]
        k = (6 * n) // 10

        binid = (lax.broadcasted_iota(jnp.int32, (HIST_R, HIST_C), 0) * HIST_C
                 + lax.broadcasted_iota(jnp.int32, (HIST_R, HIST_C), 1))

        def bs_body(_, lo_hi):
            lo, hi = lo_hi
            mid = (lo + hi) // 2
            s_mid = jnp.sum(jnp.where(binid >= mid * 8, cnt, 0))
            return jnp.where(s_mid >= k, mid, lo), jnp.where(s_mid >= k, hi, mid)

        sel, _ = lax.fori_loop(0, 13, bs_body,
                               (jnp.int32(0), jnp.int32(NBINS // 8)))
        c_hi = jnp.sum(jnp.where(binid >= (sel + 1) * 8, cnt, 0))
        m = jnp.sum(jnp.where(binid >= sel * 8, cnt, 0)) - c_hi

        s_ref[0] = sel
        s_ref[1] = c_hi
        s_ref[2] = m
        s_ref[3] = k
        f_ref[0] = 0.0
        f_ref[1] = 0.0

    sel = s_ref[0]
    bits = loss_ref[...]
    loss = lax.bitcast_convert_type(bits, jnp.float32)
    bins = lax.shift_right_logical(bits, 19)
    f_ref[0] += jnp.sum(jnp.where(bins > sel, loss, 0.0))
    f_ref[1] += jnp.sum(jnp.where(bins == sel, loss, 0.0))

    @pl.when(i == pl.num_programs(0) - 1)
    def _():
        c_hi, m, k = s_ref[1], s_ref[2], s_ref[3]
        r = jnp.clip(k - c_hi, 0, m)
        mean_in = jnp.where(m > 0, f_ref[1] / m.astype(jnp.float32), 0.0)
        out_ref[0, 0] = ((f_ref[0] + r.astype(jnp.float32) * mean_in)
                         / k.astype(jnp.float32))


def _final_pass(hist, n, loss2d):
    hist3 = hist.reshape(NW, HIST_R, HIST_C)
    return pl.pallas_call(
        _final_body,
        grid=(GRID,),
        in_specs=[
            pl.BlockSpec((NW, HIST_R, HIST_C), lambda i: (0, 0, 0)),
            pl.BlockSpec((1, 1), lambda i: (0, 0), memory_space=pltpu.SMEM),
            pl.BlockSpec((BLK_ROWS, COLS), lambda i: (i, 0)),
        ],
        out_specs=pl.BlockSpec((1, 1), lambda i: (0, 0), memory_space=pltpu.SMEM),
        out_shape=jax.ShapeDtypeStruct((1, 1), jnp.float32),
        scratch_shapes=[
            pltpu.SMEM((4,), jnp.int32),
            pltpu.SMEM((2,), jnp.float32),
        ],
    )(hist3, n, loss2d)


def kernel(inputs, targets, mask):
    loss, n = _loss_pass(inputs, targets, mask)
    hist = _hist_pass(loss)
    out = _final_pass(hist, n, loss)
    return out.reshape(())


# speedup vs baseline: 94.1569x; 1.6076x over previous
import functools

import jax
import jax.numpy as jnp
from jax import lax
from jax.experimental import pallas as pl
from jax.experimental.pallas import tpu as pltpu
from jax.experimental.pallas import tpu_sc as plsc

N_TOTAL = 64 * 512 * 512
ROWS, COLS = 32768, 512
GRID = 32
BLK3 = (2, 512, 512)
BLK_ROWS = 1024

NBINS = 65536
HIST_R, HIST_C = 512, 128

NW = 32
ROWS_W = ROWS // NW
CHUNK_R = 32
CHUNKS = ROWS_W // CHUNK_R


def _loss_body(x_ref, t_ref, m_ref, loss_ref, n_ref):
    @pl.when(pl.program_id(0) == 0)
    def _():
        n_ref[0, 0] = 0

    d = x_ref[...] - t_ref[...]
    ad = jnp.abs(d)
    sl1 = jnp.where(ad < 1.0, 0.5 * d * d, ad - 0.5)
    m = m_ref[...]
    bits = lax.bitcast_convert_type(jnp.where(m, sl1, 0.0), jnp.int32)
    loss_ref[...] = bits.reshape(BLK_ROWS, COLS)
    n_ref[0, 0] += jnp.sum(m.astype(jnp.int32))


def _loss_pass(x, t, m):
    return pl.pallas_call(
        _loss_body,
        grid=(GRID,),
        in_specs=[
            pl.BlockSpec(BLK3, lambda i: (i, 0, 0)),
            pl.BlockSpec(BLK3, lambda i: (i, 0, 0)),
            pl.BlockSpec(BLK3, lambda i: (i, 0, 0)),
        ],
        out_specs=[
            pl.BlockSpec((BLK_ROWS, COLS), lambda i: (i, 0)),
            pl.BlockSpec((1, 1), lambda i: (0, 0), memory_space=pltpu.SMEM),
        ],
        out_shape=[
            jax.ShapeDtypeStruct((ROWS, COLS), jnp.int32),
            jax.ShapeDtypeStruct((1, 1), jnp.int32),
        ],
    )(x, t, m)


def _hist_sc_body(loss_hbm, out_hbm, hist, buf0, buf1, sem0, sem1):
    wid = lax.axis_index("s") * 2 + lax.axis_index("c")
    row0 = wid * ROWS_W

    zeros16 = jnp.zeros((16,), jnp.int32)

    def zbody(i, carry):
        hist[pl.ds(i * 16, 16)] = zeros16
        return carry

    lax.fori_loop(0, NBINS // 16, zbody, 0)

    ones16 = jnp.ones((16,), jnp.int32)
    shift16 = jnp.full((16,), 16, jnp.int32)
    mask16 = jnp.full((16,), 0xFFF8, jnp.int32)
    lane3 = jnp.bitwise_and(lax.iota(jnp.int32, 16), 7)

    def chunk_at(c):
        return loss_hbm.at[pl.ds(row0 + c * CHUNK_R, CHUNK_R)]

    def process(buf):
        @plsc.parallel_loop(0, CHUNK_R, unroll=2)
        def rbody(r):
            for u in range(COLS // 16):
                bits = buf[r, pl.ds(u * 16, 16)]
                addr = jnp.bitwise_or(
                    jnp.bitwise_and(lax.shift_right_logical(bits, shift16),
                                    mask16),
                    lane3)
                plsc.addupdate_scatter(hist, [addr], ones16)

    pltpu.async_copy(chunk_at(0), buf0, sem0)

    def cbody(i, carry):
        c = 2 * i
        pltpu.async_copy(chunk_at(c + 1), buf1, sem1)
        pltpu.make_async_copy(chunk_at(c), buf0, sem0).wait()
        process(buf0)

        @pl.when(c + 2 < CHUNKS)
        def _():
            pltpu.async_copy(chunk_at(c + 2), buf0, sem0)

        pltpu.make_async_copy(chunk_at(c + 1), buf1, sem1).wait()
        process(buf1)
        return carry

    lax.fori_loop(0, CHUNKS // 2, cbody, 0)

    pltpu.sync_copy(hist, out_hbm.at[wid])


def _hist_pass(loss2d):
    mesh = plsc.VectorSubcoreMesh(core_axis_name="c", subcore_axis_name="s")
    kern = pl.kernel(
        _hist_sc_body,
        out_type=jax.ShapeDtypeStruct((NW, NBINS), jnp.int32),
        mesh=mesh,
        compiler_params=pltpu.CompilerParams(
            needs_layout_passes=False, use_tc_tiling_on_sc=False),
        scratch_types=[
            pltpu.VMEM((NBINS,), jnp.int32),
            pltpu.VMEM((CHUNK_R, COLS), jnp.int32),
            pltpu.VMEM((CHUNK_R, COLS), jnp.int32),
            pltpu.SemaphoreType.DMA,
            pltpu.SemaphoreType.DMA,
        ],
    )
    return kern(loss2d)


def _final_body(hist_ref, n_ref, loss_ref, out_ref, s_ref, f_ref):
    i = pl.program_id(0)

    @pl.when(i == 0)
    def _():
        cnt = jnp.sum(hist_ref[...], axis=0).astype(jnp.int32)
        n = n_ref[0, 0]
        k = (6 * n) // 10

        binid = (lax.broadcasted_iota(jnp.int32, (HIST_R, HIST_C), 0) * HIST_C
                 + lax.broadcasted_iota(jnp.int32, (HIST_R, HIST_C), 1))

        def bs_body(_, lo_hi):
            lo, hi = lo_hi
            mid = (lo + hi) // 2
            s_mid = jnp.sum(jnp.where(binid >= mid * 8, cnt, 0))
            return jnp.where(s_mid >= k, mid, lo), jnp.where(s_mid >= k, hi, mid)

        sel, _ = lax.fori_loop(0, 13, bs_body,
                               (jnp.int32(0), jnp.int32(NBINS // 8)))
        c_hi = jnp.sum(jnp.where(binid >= (sel + 1) * 8, cnt, 0))
        m = jnp.sum(jnp.where(binid >= sel * 8, cnt, 0)) - c_hi

        s_ref[0] = sel
        s_ref[1] = c_hi
        s_ref[2] = m
        s_ref[3] = k
        f_ref[0] = 0.0
        f_ref[1] = 0.0

    sel = s_ref[0]
    bits = loss_ref[...]
    loss = lax.bitcast_convert_type(bits, jnp.float32)
    bins = lax.shift_right_logical(bits, 19)
    f_ref[0] += jnp.sum(jnp.where(bins > sel, loss, 0.0))
    f_ref[1] += jnp.sum(jnp.where(bins == sel, loss, 0.0))

    @pl.when(i == pl.num_programs(0) - 1)
    def _():
        c_hi, m, k = s_ref[1], s_ref[2], s_ref[3]
        r = jnp.clip(k - c_hi, 0, m)
        mean_in = jnp.where(m > 0, f_ref[1] / m.astype(jnp.float32), 0.0)
        out_ref[0, 0] = ((f_ref[0] + r.astype(jnp.float32) * mean_in)
                         / k.astype(jnp.float32))


def _final_pass(hist, n, loss2d):
    hist3 = hist.reshape(NW, HIST_R, HIST_C)
    return pl.pallas_call(
        _final_body,
        grid=(GRID,),
        in_specs=[
            pl.BlockSpec((NW, HIST_R, HIST_C), lambda i: (0, 0, 0)),
            pl.BlockSpec((1, 1), lambda i: (0, 0), memory_space=pltpu.SMEM),
            pl.BlockSpec((BLK_ROWS, COLS), lambda i: (i, 0)),
        ],
        out_specs=pl.BlockSpec((1, 1), lambda i: (0, 0), memory_space=pltpu.SMEM),
        out_shape=jax.ShapeDtypeStruct((1, 1), jnp.float32),
        scratch_shapes=[
            pltpu.SMEM((4,), jnp.int32),
            pltpu.SMEM((2,), jnp.float32),
        ],
    )(hist3, n, loss2d)


def kernel(inputs, targets, mask):
    loss, n = _loss_pass(inputs, targets, mask)
    hist = _hist_pass(loss)
    out = _final_pass(hist, n, loss)
    return out.reshape(())


# speedup vs baseline: 95.0690x; 1.0097x over previous
import functools

import jax
import jax.numpy as jnp
from jax import lax
from jax.experimental import pallas as pl
from jax.experimental.pallas import tpu as pltpu
from jax.experimental.pallas import tpu_sc as plsc

N_TOTAL = 64 * 512 * 512
ROWS, COLS = 32768, 512
GRID_L = 64
BLK3 = (1, 512, 512)
BLK_L = 512
GRID = 32
BLK_ROWS = 1024

NBINS = 65536
HIST_R, HIST_C = 512, 128

NW = 32
ROWS_W = ROWS // NW
CHUNK_R = 32
CHUNKS = ROWS_W // CHUNK_R


def _loss_body(x_ref, t_ref, m_ref, loss_ref, n_ref):
    @pl.when(pl.program_id(0) == 0)
    def _():
        n_ref[0, 0] = 0

    d = x_ref[...] - t_ref[...]
    ad = jnp.abs(d)
    sl1 = jnp.where(ad < 1.0, 0.5 * d * d, ad - 0.5)
    m = m_ref[...] != 0
    bits = lax.bitcast_convert_type(jnp.where(m, sl1, 0.0), jnp.int32)
    loss_ref[...] = bits.reshape(BLK_L, COLS)
    n_ref[0, 0] += jnp.sum(m.astype(jnp.int32))


def _loss_pass(x, t, m):
    return pl.pallas_call(
        _loss_body,
        grid=(GRID_L,),
        in_specs=[
            pl.BlockSpec(BLK3, lambda i: (i, 0, 0)),
            pl.BlockSpec(BLK3, lambda i: (i, 0, 0)),
            pl.BlockSpec(BLK3, lambda i: (i, 0, 0)),
        ],
        out_specs=[
            pl.BlockSpec((BLK_L, COLS), lambda i: (i, 0)),
            pl.BlockSpec((1, 1), lambda i: (0, 0), memory_space=pltpu.SMEM),
        ],
        out_shape=[
            jax.ShapeDtypeStruct((ROWS, COLS), jnp.int32),
            jax.ShapeDtypeStruct((1, 1), jnp.int32),
        ],
    )(x, t, m.view(jnp.int8))


def _hist_sc_body(loss_hbm, out_hbm, hist, buf0, buf1, sem0, sem1):
    wid = lax.axis_index("s") * 2 + lax.axis_index("c")
    row0 = wid * ROWS_W

    zeros16 = jnp.zeros((16,), jnp.int32)

    def zbody(i, carry):
        hist[pl.ds(i * 16, 16)] = zeros16
        return carry

    lax.fori_loop(0, NBINS // 16, zbody, 0)

    ones16 = jnp.ones((16,), jnp.int32)
    shift16 = jnp.full((16,), 16, jnp.int32)
    mask16 = jnp.full((16,), 0xFFF8, jnp.int32)
    lane3 = jnp.bitwise_and(lax.iota(jnp.int32, 16), 7)

    def chunk_at(c):
        return loss_hbm.at[pl.ds(row0 + c * CHUNK_R, CHUNK_R)]

    def process(buf):
        @plsc.parallel_loop(0, CHUNK_R, unroll=2)
        def rbody(r):
            for u in range(COLS // 16):
                bits = buf[r, pl.ds(u * 16, 16)]
                addr = jnp.bitwise_or(
                    jnp.bitwise_and(lax.shift_right_logical(bits, shift16),
                                    mask16),
                    lane3)
                plsc.addupdate_scatter(hist, [addr], ones16)

    pltpu.async_copy(chunk_at(0), buf0, sem0)

    def cbody(i, carry):
        c = 2 * i
        pltpu.async_copy(chunk_at(c + 1), buf1, sem1)
        pltpu.make_async_copy(chunk_at(c), buf0, sem0).wait()
        process(buf0)

        @pl.when(c + 2 < CHUNKS)
        def _():
            pltpu.async_copy(chunk_at(c + 2), buf0, sem0)

        pltpu.make_async_copy(chunk_at(c + 1), buf1, sem1).wait()
        process(buf1)
        return carry

    lax.fori_loop(0, CHUNKS // 2, cbody, 0)

    pltpu.sync_copy(hist, out_hbm.at[wid])


def _hist_pass(loss2d):
    mesh = plsc.VectorSubcoreMesh(core_axis_name="c", subcore_axis_name="s")
    kern = pl.kernel(
        _hist_sc_body,
        out_type=jax.ShapeDtypeStruct((NW, NBINS), jnp.int32),
        mesh=mesh,
        compiler_params=pltpu.CompilerParams(
            needs_layout_passes=False, use_tc_tiling_on_sc=False),
        scratch_types=[
            pltpu.VMEM((NBINS,), jnp.int32),
            pltpu.VMEM((CHUNK_R, COLS), jnp.int32),
            pltpu.VMEM((CHUNK_R, COLS), jnp.int32),
            pltpu.SemaphoreType.DMA,
            pltpu.SemaphoreType.DMA,
        ],
    )
    return kern(loss2d)


def _final_body(hist_ref, n_ref, loss_ref, out_ref, s_ref, f_ref):
    i = pl.program_id(0)

    @pl.when(i == 0)
    def _():
        cnt = jnp.sum(hist_ref[...], axis=0).astype(jnp.int32)
        n = n_ref[0, 0]
        k = (6 * n) // 10

        binid = (lax.broadcasted_iota(jnp.int32, (HIST_R, HIST_C), 0) * HIST_C
                 + lax.broadcasted_iota(jnp.int32, (HIST_R, HIST_C), 1))

        def bs_body(_, lo_hi):
            lo, hi = lo_hi
            mid = (lo + hi) // 2
            s_mid = jnp.sum(jnp.where(binid >= mid * 8, cnt, 0))
            return jnp.where(s_mid >= k, mid, lo), jnp.where(s_mid >= k, hi, mid)

        sel, _ = lax.fori_loop(0, 13, bs_body,
                               (jnp.int32(0), jnp.int32(NBINS // 8)))
        c_hi = jnp.sum(jnp.where(binid >= (sel + 1) * 8, cnt, 0))
        m = jnp.sum(jnp.where(binid >= sel * 8, cnt, 0)) - c_hi

        s_ref[0] = sel
        s_ref[1] = c_hi
        s_ref[2] = m
        s_ref[3] = k
        f_ref[0] = 0.0
        f_ref[1] = 0.0

    sel = s_ref[0]
    bits = loss_ref[...]
    loss = lax.bitcast_convert_type(bits, jnp.float32)
    bins = lax.shift_right_logical(bits, 19)
    f_ref[0] += jnp.sum(jnp.where(bins > sel, loss, 0.0))
    f_ref[1] += jnp.sum(jnp.where(bins == sel, loss, 0.0))

    @pl.when(i == pl.num_programs(0) - 1)
    def _():
        c_hi, m, k = s_ref[1], s_ref[2], s_ref[3]
        r = jnp.clip(k - c_hi, 0, m)
        mean_in = jnp.where(m > 0, f_ref[1] / m.astype(jnp.float32), 0.0)
        out_ref[0, 0] = ((f_ref[0] + r.astype(jnp.float32) * mean_in)
                         / k.astype(jnp.float32))


def _final_pass(hist, n, loss2d):
    hist3 = hist.reshape(NW, HIST_R, HIST_C)
    return pl.pallas_call(
        _final_body,
        grid=(GRID,),
        in_specs=[
            pl.BlockSpec((NW, HIST_R, HIST_C), lambda i: (0, 0, 0)),
            pl.BlockSpec((1, 1), lambda i: (0, 0), memory_space=pltpu.SMEM),
            pl.BlockSpec((BLK_ROWS, COLS), lambda i: (i, 0)),
        ],
        out_specs=pl.BlockSpec((1, 1), lambda i: (0, 0), memory_space=pltpu.SMEM),
        out_shape=jax.ShapeDtypeStruct((1, 1), jnp.float32),
        scratch_shapes=[
            pltpu.SMEM((4,), jnp.int32),
            pltpu.SMEM((2,), jnp.float32),
        ],
    )(hist3, n, loss2d)


def kernel(inputs, targets, mask):
    loss, n = _loss_pass(inputs, targets, mask)
    hist = _hist_pass(loss)
    out = _final_pass(hist, n, loss)
    return out.reshape(())


# speedup vs baseline: 108.7635x; 1.1440x over previous
import functools

import jax
import jax.numpy as jnp
from jax import lax
from jax.experimental import pallas as pl
from jax.experimental.pallas import tpu as pltpu
from jax.experimental.pallas import tpu_sc as plsc

N_TOTAL = 64 * 512 * 512
ROWS, COLS = 32768, 512
GRID_L = 32
BLK3 = (2, 512, 512)
BLK_L = 1024
GRID = 32
BLK_ROWS = 1024

NBINS = 65536
HIST_R, HIST_C = 512, 128

NW = 32
ROWS_W = ROWS // NW
CHUNK_R = 32
CHUNKS = ROWS_W // CHUNK_R


def _loss_body(x_ref, t_ref, m_ref, loss_ref, n_ref):
    @pl.when(pl.program_id(0) == 0)
    def _():
        n_ref[0, 0] = 0

    d = x_ref[...] - t_ref[...]
    ad = jnp.abs(d)
    sl1 = jnp.where(ad < 1.0, 0.5 * d * d, ad - 0.5)
    m = m_ref[...] != 0
    bits = lax.bitcast_convert_type(jnp.where(m, sl1, 0.0), jnp.int32)
    loss_ref[...] = bits.reshape(BLK_L, COLS)
    n_ref[0, 0] += jnp.sum(m.astype(jnp.int32))


def _loss_pass(x, t, m):
    return pl.pallas_call(
        _loss_body,
        grid=(GRID_L,),
        in_specs=[
            pl.BlockSpec(BLK3, lambda i: (i, 0, 0)),
            pl.BlockSpec(BLK3, lambda i: (i, 0, 0)),
            pl.BlockSpec(BLK3, lambda i: (i, 0, 0)),
        ],
        out_specs=[
            pl.BlockSpec((BLK_L, COLS), lambda i: (i, 0)),
            pl.BlockSpec((1, 1), lambda i: (0, 0), memory_space=pltpu.SMEM),
        ],
        out_shape=[
            jax.ShapeDtypeStruct((ROWS, COLS), jnp.int32),
            jax.ShapeDtypeStruct((1, 1), jnp.int32),
        ],
    )(x, t, m.view(jnp.int8))


def _hist_sc_body(loss_hbm, out_hbm, hist, buf0, buf1, sem0, sem1):
    wid = lax.axis_index("s") * 2 + lax.axis_index("c")
    row0 = wid * ROWS_W

    zeros16 = jnp.zeros((16,), jnp.int32)

    def zbody(i, carry):
        hist[pl.ds(i * 16, 16)] = zeros16
        return carry

    lax.fori_loop(0, NBINS // 16, zbody, 0)

    ones16 = jnp.ones((16,), jnp.int32)
    shift16 = jnp.full((16,), 16, jnp.int32)
    mask16 = jnp.full((16,), 0xFFF8, jnp.int32)
    lane3 = jnp.bitwise_and(lax.iota(jnp.int32, 16), 7)

    def chunk_at(c):
        return loss_hbm.at[pl.ds(row0 + c * CHUNK_R, CHUNK_R)]

    def process(buf):
        @plsc.parallel_loop(0, CHUNK_R, unroll=4)
        def rbody(r):
            for u in range(COLS // 16):
                bits = buf[r, pl.ds(u * 16, 16)]
                addr = jnp.bitwise_or(
                    jnp.bitwise_and(lax.shift_right_logical(bits, shift16),
                                    mask16),
                    lane3)
                plsc.addupdate_scatter(hist, [addr], ones16)

    pltpu.async_copy(chunk_at(0), buf0, sem0)

    def cbody(i, carry):
        c = 2 * i
        pltpu.async_copy(chunk_at(c + 1), buf1, sem1)
        pltpu.make_async_copy(chunk_at(c), buf0, sem0).wait()
        process(buf0)

        @pl.when(c + 2 < CHUNKS)
        def _():
            pltpu.async_copy(chunk_at(c + 2), buf0, sem0)

        pltpu.make_async_copy(chunk_at(c + 1), buf1, sem1).wait()
        process(buf1)
        return carry

    lax.fori_loop(0, CHUNKS // 2, cbody, 0)

    pltpu.sync_copy(hist, out_hbm.at[wid])


def _hist_pass(loss2d):
    mesh = plsc.VectorSubcoreMesh(core_axis_name="c", subcore_axis_name="s")
    kern = pl.kernel(
        _hist_sc_body,
        out_type=jax.ShapeDtypeStruct((NW, NBINS), jnp.int32),
        mesh=mesh,
        compiler_params=pltpu.CompilerParams(
            needs_layout_passes=False, use_tc_tiling_on_sc=True),
        scratch_types=[
            pltpu.VMEM((NBINS,), jnp.int32),
            pltpu.VMEM((CHUNK_R, COLS), jnp.int32),
            pltpu.VMEM((CHUNK_R, COLS), jnp.int32),
            pltpu.SemaphoreType.DMA,
            pltpu.SemaphoreType.DMA,
        ],
    )
    return kern(loss2d)


def _final_body(hist_ref, n_ref, loss_ref, out_ref, s_ref, f_ref):
    i = pl.program_id(0)

    @pl.when(i == 0)
    def _():
        cnt = jnp.sum(hist_ref[...], axis=0).astype(jnp.int32)
        n = n_ref[0, 0]
        k = (6 * n) // 10

        binid = (lax.broadcasted_iota(jnp.int32, (HIST_R, HIST_C), 0) * HIST_C
                 + lax.broadcasted_iota(jnp.int32, (HIST_R, HIST_C), 1))

        def bs_body(_, lo_hi):
            lo, hi = lo_hi
            mid = (lo + hi) // 2
            s_mid = jnp.sum(jnp.where(binid >= mid * 8, cnt, 0))
            return jnp.where(s_mid >= k, mid, lo), jnp.where(s_mid >= k, hi, mid)

        sel, _ = lax.fori_loop(0, 13, bs_body,
                               (jnp.int32(0), jnp.int32(NBINS // 8)))
        c_hi = jnp.sum(jnp.where(binid >= (sel + 1) * 8, cnt, 0))
        m = jnp.sum(jnp.where(binid >= sel * 8, cnt, 0)) - c_hi

        s_ref[0] = sel
        s_ref[1] = c_hi
        s_ref[2] = m
        s_ref[3] = k
        f_ref[0] = 0.0
        f_ref[1] = 0.0

    sel = s_ref[0]
    bits = loss_ref[...]
    loss = lax.bitcast_convert_type(bits, jnp.float32)
    bins = lax.shift_right_logical(bits, 19)
    f_ref[0] += jnp.sum(jnp.where(bins > sel, loss, 0.0))
    f_ref[1] += jnp.sum(jnp.where(bins == sel, loss, 0.0))

    @pl.when(i == pl.num_programs(0) - 1)
    def _():
        c_hi, m, k = s_ref[1], s_ref[2], s_ref[3]
        r = jnp.clip(k - c_hi, 0, m)
        mean_in = jnp.where(m > 0, f_ref[1] / m.astype(jnp.float32), 0.0)
        out_ref[0, 0] = ((f_ref[0] + r.astype(jnp.float32) * mean_in)
                         / k.astype(jnp.float32))


def _final_pass(hist, n, loss2d):
    hist3 = hist.reshape(NW, HIST_R, HIST_C)
    return pl.pallas_call(
        _final_body,
        grid=(GRID,),
        in_specs=[
            pl.BlockSpec((NW, HIST_R, HIST_C), lambda i: (0, 0, 0)),
            pl.BlockSpec((1, 1), lambda i: (0, 0), memory_space=pltpu.SMEM),
            pl.BlockSpec((BLK_ROWS, COLS), lambda i: (i, 0)),
        ],
        out_specs=pl.BlockSpec((1, 1), lambda i: (0, 0), memory_space=pltpu.SMEM),
        out_shape=jax.ShapeDtypeStruct((1, 1), jnp.float32),
        scratch_shapes=[
            pltpu.SMEM((4,), jnp.int32),
            pltpu.SMEM((2,), jnp.float32),
        ],
    )(hist3, n, loss2d)


def kernel(inputs, targets, mask):
    loss, n = _loss_pass(inputs, targets, mask)
    hist = _hist_pass(loss)
    out = _final_pass(hist, n, loss)
    return out.reshape(())
